# baseline (speedup 1.0000x reference)
import functools

import jax
import jax.numpy as jnp
from jax import lax
from jax.experimental import pallas as pl
from jax.experimental.pallas import tpu as pltpu
from jax.experimental.pallas import tpu_sc as plsc

DIM = 32
N_ROWS = 4096 * 50
NC, NS, L = 2, 16, 16
NW = NC * NS
R = N_ROWS // NW
NB = 1280
NBLK = R // NB
GCH = 128
NG = NB // GCH


def _embed_body(x_hbm, w_hbm, out_hbm, xb, idxp, rows, pos, sem):
    wid = lax.axis_index("s") * NC + lax.axis_index("c")
    base = wid * R

    pltpu.sync_copy(x_hbm.at[pl.ds(base, R)], xb)

    zero16 = jnp.zeros((L,), jnp.float32)
    lane = lax.iota(jnp.int32, L)

    for blk in range(NBLK):
        b0 = blk * NB

        def transform(j, cnt):
            off = b0 + j * L
            x16 = xb[pl.ds(off, L)]
            m = x16 == 0
            idxp[pl.ds(off, L)] = jnp.maximum(x16 - 1, 0)
            posv = jnp.full((L,), j * L, jnp.int32) + lane
            mi = m.astype(jnp.int32)
            dst = cnt + plsc.cumsum(mi) - 1
            plsc.store_scatter(pos, [dst], posv, mask=m)
            pc = plsc.all_reduce_population_count(m)
            return cnt + pc[0]

        cnt = lax.fori_loop(0, NB // L, transform, jnp.int32(0))

        cps = []
        for c in range(NG):
            cps.append(pltpu.async_copy(
                w_hbm.at[idxp.at[pl.ds(b0 + c * GCH, GCH)]],
                rows.at[pl.ds(c * GCH, GCH)],
                sem,
            ))
        for cp in cps:
            cp.wait()

        def fixup(t, carry):
            r = pos[pl.ds(t, L)][0]
            rows[r, pl.ds(0, L)] = zero16
            rows[r, pl.ds(L, L)] = zero16
            return carry

        lax.fori_loop(0, cnt, fixup, jnp.int32(0))

        pltpu.sync_copy(rows, out_hbm.at[pl.ds(base + b0, NB)])


@functools.partial(jax.jit, static_argnums=())
def _embed(xf, W):
    mesh = plsc.VectorSubcoreMesh(core_axis_name="c", subcore_axis_name="s")
    fn = functools.partial(
        pl.kernel,
        mesh=mesh,
        out_type=jax.ShapeDtypeStruct((N_ROWS, DIM), jnp.float32),
        scratch_types=[
            pltpu.VMEM((R,), jnp.int32),
            pltpu.VMEM((R,), jnp.int32),
            pltpu.VMEM((NB, DIM), jnp.float32),
            pltpu.VMEM((NB + L,), jnp.int32),
            pltpu.SemaphoreType.DMA,
        ],
        compiler_params=pltpu.CompilerParams(use_tc_tiling_on_sc=False, needs_layout_passes=False),
    )(_embed_body)
    return fn(xf, W)


def kernel(x, W):
    xf = x.reshape(-1).astype(jnp.int32)
    out = _embed(xf, W)
    return out.reshape(x.shape + (W.shape[1],))


# speedup vs baseline: 1.0004x; 1.0004x over previous
import functools

import jax
import jax.numpy as jnp
from jax import lax
from jax.experimental import pallas as pl
from jax.experimental.pallas import tpu as pltpu
from jax.experimental.pallas import tpu_sc as plsc

DIM = 32
N_ROWS = 4096 * 50
NC, NS, L = 2, 16, 16
NW = NC * NS
R = N_ROWS // NW
NB = 1280
NBLK = R // NB
GCH = 128
NG = NB // GCH


def _embed_body(x_hbm, w_hbm, out_hbm, xb, idxp, rows, pos, sem):
    wid = lax.axis_index("s") * NC + lax.axis_index("c")
    base = wid * R

    pltpu.sync_copy(x_hbm.at[pl.ds(base, R)], xb)

    zero16 = jnp.zeros((L,), jnp.float32)
    lane = lax.iota(jnp.int32, L)

    for blk in range(NBLK):
        b0 = blk * NB

        def transform(j, cnt):
            off = b0 + j * L
            x16 = xb[pl.ds(off, L)]
            m = x16 == 0
            idxp[pl.ds(off, L)] = jnp.maximum(x16 - 1, 0)
            posv = jnp.full((L,), j * L, jnp.int32) + lane
            mi = m.astype(jnp.int32)
            dst = cnt + plsc.cumsum(mi) - 1
            plsc.store_scatter(pos, [dst], posv, mask=m)
            pc = plsc.all_reduce_population_count(m)
            return cnt + pc[0]

        cnt = lax.fori_loop(0, NB // L, transform, jnp.int32(0))

        cps = []
        for c in range(NG):
            cps.append(pltpu.async_copy(
                w_hbm.at[idxp.at[pl.ds(b0 + c * GCH, GCH)]],
                rows.at[pl.ds(c * GCH, GCH)],
                sem,
            ))
        for cp in cps:
            cp.wait()

        def fixup(t, carry):
            r = pos[pl.ds(t, L)][0]
            rows[r, pl.ds(0, L)] = zero16
            rows[r, pl.ds(L, L)] = zero16
            return carry

        lax.fori_loop(0, cnt, fixup, jnp.int32(0))

        pltpu.sync_copy(rows, out_hbm.at[pl.ds(base + b0, NB)])


@functools.partial(jax.jit, static_argnums=())
def _embed(xf, w1):
    W = w1.reshape(-1, DIM)
    mesh = plsc.VectorSubcoreMesh(core_axis_name="c", subcore_axis_name="s")
    fn = functools.partial(
        pl.kernel,
        mesh=mesh,
        out_type=jax.ShapeDtypeStruct((N_ROWS, DIM), jnp.float32),
        scratch_types=[
            pltpu.VMEM((R,), jnp.int32),
            pltpu.VMEM((R,), jnp.int32),
            pltpu.VMEM((NB, DIM), jnp.float32),
            pltpu.VMEM((NB + L,), jnp.int32),
            pltpu.SemaphoreType.DMA,
        ],
        compiler_params=pltpu.CompilerParams(use_tc_tiling_on_sc=False, needs_layout_passes=False),
    )(_embed_body)
    return fn(xf, W)


def kernel(x, W):
    xf = x.reshape(-1).astype(jnp.int32)
    out = _embed(xf, W.reshape(-1))
    return out.reshape(x.shape + (W.shape[1],))


# speedup vs baseline: 1.2149x; 1.2144x over previous
import functools

import jax
import jax.numpy as jnp
from jax import lax
from jax.experimental import pallas as pl
from jax.experimental.pallas import tpu as pltpu
from jax.experimental.pallas import tpu_sc as plsc

DIM = 32
BATCH = 4096
HIST = 50
NC, NS, L = 2, 16, 16
NW = NC * NS
BPW = BATCH // NW
BB = 16
NBLK = BPW // BB
NB = BB * HIST


def _embed_body(xt_hbm, w_hbm, out_hbm, xb, idxp, rows, pos, sem):
    wid = lax.axis_index("s") * NC + lax.axis_index("c")
    bstart = wid * BPW

    pltpu.sync_copy(xt_hbm.at[:, pl.ds(bstart, BPW)], xb)

    zero16 = jnp.zeros((L,), jnp.float32)
    lane = lax.iota(jnp.int32, L)

    for blk in range(NBLK):
        bl0 = blk * BB

        def transform(h, cnt):
            v = xb[h, pl.ds(bl0, L)]
            m = v == 0
            rows_i = jnp.full((L,), bl0, jnp.int32) + lane
            cols_i = jnp.full((L,), h, jnp.int32)
            plsc.store_scatter(idxp, [rows_i, cols_i], jnp.maximum(v - 1, 0))
            pc = plsc.all_reduce_population_count(m)
            nz = pc[0]

            @pl.when(nz > 0)
            def _():
                posv = lane * HIST + cols_i
                dst = cnt + plsc.cumsum(m.astype(jnp.int32)) - 1
                plsc.store_scatter(pos, [dst], posv, mask=m)

            return cnt + nz

        cnt = lax.fori_loop(0, HIST, transform, jnp.int32(0))

        cps = []
        for rb in range(BB):
            cps.append(pltpu.async_copy(
                w_hbm.at[idxp.at[bl0 + rb]],
                rows.at[rb],
                sem,
            ))
        for cp in cps:
            cp.wait()

        def fixup(t, carry):
            p = pos[pl.ds(t, L)][0]
            rb = p // HIST
            rh = p - rb * HIST
            rows[rb, rh, pl.ds(0, L)] = zero16
            rows[rb, rh, pl.ds(L, L)] = zero16
            return carry

        lax.fori_loop(0, cnt, fixup, jnp.int32(0))

        pltpu.sync_copy(rows, out_hbm.at[pl.ds(bstart + bl0, BB), :, :])


@functools.partial(jax.jit, static_argnums=())
def _embed(xt, W):
    mesh = plsc.VectorSubcoreMesh(core_axis_name="c", subcore_axis_name="s")
    fn = functools.partial(
        pl.kernel,
        mesh=mesh,
        out_type=jax.ShapeDtypeStruct((BATCH, HIST, DIM), jnp.float32),
        scratch_types=[
            pltpu.VMEM((HIST, BPW), jnp.int32),
            pltpu.VMEM((BPW, HIST), jnp.int32),
            pltpu.VMEM((BB, HIST, DIM), jnp.float32),
            pltpu.VMEM((NB + L,), jnp.int32),
            pltpu.SemaphoreType.DMA,
        ],
        compiler_params=pltpu.CompilerParams(
            use_tc_tiling_on_sc=False, needs_layout_passes=False),
    )(_embed_body)
    return fn(xt, W)


def kernel(x, W):
    return _embed(x.T, W)
